# baseline (speedup 1.0000x reference)
import functools

import jax
import jax.numpy as jnp
from jax import lax
from jax.experimental import pallas as pl
from jax.experimental.pallas import tpu as pltpu
from jax.experimental.pallas import tpu_sc as plsc

_N_NODES = 10000
_N_EDGES = 320000
_D = 128

_NC = 2
_NS = 16
_NW = _NC * _NS
_L = 16
_CH = 128
_NCH = _N_EDGES // _CH
_CPW = -(-_NCH // _NW)


def _pq_body(z_ref, w_ref, b_ref, p_ref, q_ref):
    w2 = jnp.concatenate([w_ref[0:1, :_D], w_ref[0:1, _D:]], axis=0)
    pq = lax.dot_general(
        w2, z_ref[...], (((1,), (1,)), ((), ())),
        preferred_element_type=jnp.float32,
    )
    p_ref[...] = -(pq[0] + b_ref[0])
    q_ref[...] = -pq[1]


_mesh = plsc.VectorSubcoreMesh(core_axis_name="c", subcore_axis_name="s")


@functools.partial(
    pl.kernel,
    out_type=jax.ShapeDtypeStruct((_N_EDGES,), jnp.float32),
    mesh=_mesh,
    compiler_params=pltpu.CompilerParams(
        needs_layout_passes=False,
        use_tc_tiling_on_sc=False,
    ),
    scratch_types=[
        pltpu.VMEM((_N_NODES,), jnp.float32),
        pltpu.VMEM((_N_NODES,), jnp.float32),
        pltpu.VMEM((_CPW * 2 * _CH,), jnp.int32),
        pltpu.VMEM((_CPW * _CH,), jnp.float32),
        pltpu.SemaphoreType.DMA,
    ],
)
def _edge_sigmoid(p_hbm, q_hbm, ei_hbm, out_hbm,
                  p_v, q_v, ei_v, o_v, sem):
    wid = lax.axis_index("s") * _NC + lax.axis_index("c")
    cb = jnp.minimum((_NCH // _NW) * wid + jnp.minimum(wid, _NCH % _NW),
                     _NCH - _CPW)
    c1 = pltpu.async_copy(p_hbm, p_v, sem)
    c2 = pltpu.async_copy(q_hbm, q_v, sem)
    c3 = pltpu.async_copy(ei_hbm.at[pl.ds(cb * 2 * _CH, _CPW * 2 * _CH)],
                          ei_v, sem)
    c1.wait()
    c2.wait()
    c3.wait()

    @plsc.parallel_loop(0, _CPW, step=1, unroll=1)
    def _loop(k):
        for g in range(_CH // _L):
            sv = ei_v[pl.ds(k * 2 * _CH + g * _L, _L)]
            dv = ei_v[pl.ds(k * 2 * _CH + _CH + g * _L, _L)]
            pv = plsc.load_gather(p_v, [sv])
            qv = plsc.load_gather(q_v, [dv])
            o_v[pl.ds(k * _CH + g * _L, _L)] = 1.0 / (1.0 + jnp.exp(pv + qv))

    pltpu.sync_copy(o_v, out_hbm.at[pl.ds(cb * _CH, _CPW * _CH)])


def kernel(z, edge_index, W, b):
    ei = (edge_index.astype(jnp.int32)
          .reshape(2, _NCH, _CH).transpose(1, 0, 2).reshape(-1))
    p, q = pl.pallas_call(
        _pq_body,
        out_shape=[
            jax.ShapeDtypeStruct((_N_NODES,), jnp.float32),
            jax.ShapeDtypeStruct((_N_NODES,), jnp.float32),
        ],
        in_specs=[
            pl.BlockSpec(memory_space=pltpu.VMEM),
            pl.BlockSpec(memory_space=pltpu.VMEM),
            pl.BlockSpec(memory_space=pltpu.SMEM),
        ],
    )(z, W, b)
    return _edge_sigmoid(p, q, ei)


# speedup vs baseline: 1.4744x; 1.4744x over previous
import functools

import jax
import jax.numpy as jnp
from jax import lax
from jax.experimental import pallas as pl
from jax.experimental.pallas import tpu as pltpu
from jax.experimental.pallas import tpu_sc as plsc

_N_NODES = 10000
_N_EDGES = 320000
_D = 128

_NC = 2
_NS = 16
_NW = _NC * _NS
_L = 16
_CH = 128
_NCH = _N_EDGES // _CH
_CPW = -(-_NCH // _NW)


def _pq_body(z_ref, w_ref, b_ref, p_ref, q_ref):
    w2 = jnp.concatenate([w_ref[0:1, :_D], w_ref[0:1, _D:]], axis=0)
    pq = lax.dot_general(
        w2, z_ref[...], (((1,), (1,)), ((), ())),
        preferred_element_type=jnp.float32,
    )
    p_ref[...] = -(pq[0] + b_ref[0])
    q_ref[...] = -pq[1]


_mesh = plsc.VectorSubcoreMesh(core_axis_name="c", subcore_axis_name="s")


@functools.partial(
    pl.kernel,
    out_type=jax.ShapeDtypeStruct((_N_EDGES,), jnp.float32),
    mesh=_mesh,
    compiler_params=pltpu.CompilerParams(
        needs_layout_passes=False,
        use_tc_tiling_on_sc=True,
    ),
    scratch_types=[
        pltpu.VMEM((_N_NODES,), jnp.float32),
        pltpu.VMEM((_N_NODES,), jnp.float32),
        pltpu.VMEM((2, _CPW * _CH), jnp.int32),
        pltpu.VMEM((_CPW * _CH,), jnp.float32),
        pltpu.SemaphoreType.DMA,
    ],
)
def _edge_sigmoid(p_hbm, q_hbm, ei_hbm, out_hbm,
                  p_v, q_v, ei_v, o_v, sem):
    wid = lax.axis_index("s") * _NC + lax.axis_index("c")
    cb = jnp.minimum((_NCH // _NW) * wid + jnp.minimum(wid, _NCH % _NW),
                     _NCH - _CPW)
    c1 = pltpu.async_copy(p_hbm, p_v, sem)
    c2 = pltpu.async_copy(q_hbm, q_v, sem)
    c3 = pltpu.async_copy(ei_hbm.at[:, pl.ds(cb * _CH, _CPW * _CH)],
                          ei_v, sem)
    c1.wait()
    c2.wait()
    c3.wait()

    @plsc.parallel_loop(0, _CPW * _CH, step=_L, unroll=2)
    def _loop(off):
        sv = ei_v[0, pl.ds(off, _L)]
        dv = ei_v[1, pl.ds(off, _L)]
        pv = plsc.load_gather(p_v, [sv])
        qv = plsc.load_gather(q_v, [dv])
        o_v[pl.ds(off, _L)] = 1.0 / (1.0 + jnp.exp(pv + qv))

    pltpu.sync_copy(o_v, out_hbm.at[pl.ds(cb * _CH, _CPW * _CH)])


def kernel(z, edge_index, W, b):
    ei = edge_index.astype(jnp.int32)
    p, q = pl.pallas_call(
        _pq_body,
        out_shape=[
            jax.ShapeDtypeStruct((_N_NODES,), jnp.float32),
            jax.ShapeDtypeStruct((_N_NODES,), jnp.float32),
        ],
        in_specs=[
            pl.BlockSpec(memory_space=pltpu.VMEM),
            pl.BlockSpec(memory_space=pltpu.VMEM),
            pl.BlockSpec(memory_space=pltpu.SMEM),
        ],
    )(z, W, b)
    return _edge_sigmoid(p, q, ei)


# speedup vs baseline: 1.4786x; 1.0029x over previous
import functools

import jax
import jax.numpy as jnp
from jax import lax
from jax.experimental import pallas as pl
from jax.experimental.pallas import tpu as pltpu
from jax.experimental.pallas import tpu_sc as plsc

_N_NODES = 10000
_N_EDGES = 320000
_D = 128

_NC = 2
_NS = 16
_NW = _NC * _NS
_L = 16
_CH = 128
_NCH = _N_EDGES // _CH
_CPW = -(-_NCH // _NW)


def _pq_body(z_ref, w_ref, b_ref, p_ref, q_ref):
    w2 = jnp.concatenate([w_ref[0:1, :_D], w_ref[0:1, _D:]], axis=0)
    pq = lax.dot_general(
        w2, z_ref[...], (((1,), (1,)), ((), ())),
        preferred_element_type=jnp.float32,
    )
    p_ref[...] = -(pq[0] + b_ref[0])
    q_ref[...] = -pq[1]


_mesh = plsc.VectorSubcoreMesh(core_axis_name="c", subcore_axis_name="s")


@functools.partial(
    pl.kernel,
    out_type=jax.ShapeDtypeStruct((_N_EDGES,), jnp.float32),
    mesh=_mesh,
    compiler_params=pltpu.CompilerParams(
        needs_layout_passes=False,
        use_tc_tiling_on_sc=True,
    ),
    scratch_types=[
        pltpu.VMEM((_N_NODES,), jnp.float32),
        pltpu.VMEM((_N_NODES,), jnp.float32),
        pltpu.VMEM((2, _CPW * _CH), jnp.int32),
        pltpu.VMEM((_CPW * _CH,), jnp.float32),
        pltpu.SemaphoreType.DMA,
    ],
)
def _edge_sigmoid(p_hbm, q_hbm, ei_hbm, out_hbm,
                  p_v, q_v, ei_v, o_v, sem):
    wid = lax.axis_index("s") * _NC + lax.axis_index("c")
    cb = jnp.minimum((_NCH // _NW) * wid + jnp.minimum(wid, _NCH % _NW),
                     _NCH - _CPW)
    c1 = pltpu.async_copy(p_hbm, p_v, sem)
    c2 = pltpu.async_copy(q_hbm, q_v, sem)
    c3 = pltpu.async_copy(ei_hbm.at[:, pl.ds(cb * _CH, _CPW * _CH)],
                          ei_v, sem)
    c1.wait()
    c2.wait()
    c3.wait()

    @plsc.parallel_loop(0, _CPW * _CH, step=_L, unroll=4)
    def _loop(off):
        sv = ei_v[0, pl.ds(off, _L)]
        dv = ei_v[1, pl.ds(off, _L)]
        pv = plsc.load_gather(p_v, [sv])
        qv = plsc.load_gather(q_v, [dv])
        o_v[pl.ds(off, _L)] = 1.0 / (1.0 + jnp.exp(pv + qv))

    pltpu.sync_copy(o_v, out_hbm.at[pl.ds(cb * _CH, _CPW * _CH)])


def kernel(z, edge_index, W, b):
    ei = edge_index.astype(jnp.int32)
    p, q = pl.pallas_call(
        _pq_body,
        out_shape=[
            jax.ShapeDtypeStruct((_N_NODES,), jnp.float32),
            jax.ShapeDtypeStruct((_N_NODES,), jnp.float32),
        ],
        in_specs=[
            pl.BlockSpec(memory_space=pltpu.VMEM),
            pl.BlockSpec(memory_space=pltpu.VMEM),
            pl.BlockSpec(memory_space=pltpu.SMEM),
        ],
    )(z, W, b)
    return _edge_sigmoid(p, q, ei)
